# baseline (speedup 1.0000x reference)
import functools

import jax
import jax.numpy as jnp
from jax import lax
from jax.experimental import pallas as pl
from jax.experimental.pallas import tpu as pltpu
from jax.experimental.pallas import tpu_sc as plsc

_NUM_SC_CORES = 2
_NUM_SC_SUBCORES = 16
_NW = _NUM_SC_CORES * _NUM_SC_SUBCORES


def _make_gather(batch: int, emb: int):
    b_per_w = batch // _NW
    mesh = plsc.VectorSubcoreMesh(
        core_axis_name="c", subcore_axis_name="s",
        num_cores=_NUM_SC_CORES, num_subcores=_NUM_SC_SUBCORES)

    @functools.partial(
        pl.kernel,
        mesh=mesh,
        compiler_params=pltpu.CompilerParams(use_tc_tiling_on_sc=False),
        out_type=[
            jax.ShapeDtypeStruct((batch, emb), jnp.bfloat16),
            jax.ShapeDtypeStruct((batch, emb), jnp.bfloat16),
        ],
        scratch_types=[
            pltpu.VMEM((b_per_w,), jnp.int32),
            pltpu.VMEM((b_per_w,), jnp.int32),
            pltpu.VMEM((b_per_w, emb), jnp.bfloat16),
            pltpu.VMEM((b_per_w, emb), jnp.bfloat16),
            pltpu.SemaphoreType.DMA,
            pltpu.SemaphoreType.DMA,
        ],
    )
    def gather_k(uids_hbm, iids_hbm, utab_hbm, itab_hbm, uout_hbm, iout_hbm,
                 uidx_v, iidx_v, urows_v, irows_v, sem_u, sem_i):
        wid = lax.axis_index("s") * _NUM_SC_CORES + lax.axis_index("c")
        base = wid * b_per_w
        pltpu.sync_copy(uids_hbm.at[pl.ds(base, b_per_w)], uidx_v)
        pltpu.sync_copy(iids_hbm.at[pl.ds(base, b_per_w)], iidx_v)
        cu = pltpu.async_copy(utab_hbm.at[uidx_v], urows_v, sem_u)
        ci = pltpu.async_copy(itab_hbm.at[iidx_v], irows_v, sem_i)
        cu.wait()
        pltpu.sync_copy(urows_v, uout_hbm.at[pl.ds(base, b_per_w)])
        ci.wait()
        pltpu.sync_copy(irows_v, iout_hbm.at[pl.ds(base, b_per_w)])

    return gather_k


def _mlp_body(u_ref, i_ref, w1u_ref, w1i_ref, b1_ref, w2_ref, b2_ref, o_ref):
    dn = (((1,), (1,)), ((), ()))
    h = lax.dot_general(u_ref[...], w1u_ref[...], dn,
                        preferred_element_type=jnp.float32)
    h = h + lax.dot_general(i_ref[...], w1i_ref[...], dn,
                            preferred_element_type=jnp.float32)
    h = jnp.maximum(h + b1_ref[...], 0.0)
    o = lax.dot_general(h, w2_ref[...], dn, preferred_element_type=jnp.float32)
    o_ref[...] = o + b2_ref[...]


def kernel(user_ids, item_ids, user_table, item_table, W1, b1, W2, b2):
    batch = user_ids.shape[0]
    emb = user_table.shape[1]
    hidden = W1.shape[0]

    gather_k = _make_gather(batch, emb)
    u_emb, i_emb = gather_k(user_ids.astype(jnp.int32),
                            item_ids.astype(jnp.int32),
                            user_table.astype(jnp.bfloat16),
                            item_table.astype(jnp.bfloat16))

    w1u = W1[:, :emb].astype(jnp.bfloat16)
    w1i = W1[:, emb:].astype(jnp.bfloat16)
    b1r = b1.reshape(1, hidden)
    b2r = b2.reshape(1, hidden)

    bm = 2048
    grid = (batch // bm,)
    out = pl.pallas_call(
        _mlp_body,
        grid=grid,
        in_specs=[
            pl.BlockSpec((bm, emb), lambda i: (i, 0)),
            pl.BlockSpec((bm, emb), lambda i: (i, 0)),
            pl.BlockSpec((hidden, emb), lambda i: (0, 0)),
            pl.BlockSpec((hidden, emb), lambda i: (0, 0)),
            pl.BlockSpec((1, hidden), lambda i: (0, 0)),
            pl.BlockSpec((hidden, hidden), lambda i: (0, 0)),
            pl.BlockSpec((1, hidden), lambda i: (0, 0)),
        ],
        out_specs=pl.BlockSpec((bm, hidden), lambda i: (i, 0)),
        out_shape=jax.ShapeDtypeStruct((batch, hidden), jnp.float32),
    )(u_emb, i_emb, w1u, w1i, b1r, W2, b2r)
    return out


# speedup vs baseline: 1.3107x; 1.3107x over previous
import functools

import jax
import jax.numpy as jnp
from jax import lax
from jax.experimental import pallas as pl
from jax.experimental.pallas import tpu as pltpu
from jax.experimental.pallas import tpu_sc as plsc

_NUM_SC_CORES = 2
_NUM_SC_SUBCORES = 16
_NW = _NUM_SC_CORES * _NUM_SC_SUBCORES


def _make_gather(batch: int, width: int):
    b_per_w = batch // _NW
    chunk = 128
    n_chunks = b_per_w // chunk
    mesh = plsc.VectorSubcoreMesh(
        core_axis_name="c", subcore_axis_name="s",
        num_cores=_NUM_SC_CORES, num_subcores=_NUM_SC_SUBCORES)

    @functools.partial(
        pl.kernel,
        mesh=mesh,
        out_type=[
            jax.ShapeDtypeStruct((batch, width), jnp.float32),
            jax.ShapeDtypeStruct((batch, width), jnp.float32),
        ],
        scratch_types=[
            pltpu.VMEM((b_per_w,), jnp.int32),
            pltpu.VMEM((b_per_w,), jnp.int32),
            pltpu.VMEM((chunk, width), jnp.float32),
            pltpu.VMEM((chunk, width), jnp.float32),
            pltpu.SemaphoreType.DMA,
            pltpu.SemaphoreType.DMA,
        ],
    )
    def gather_k(uids_hbm, iids_hbm, utab_hbm, itab_hbm, uout_hbm, iout_hbm,
                 uidx_v, iidx_v, buf0, buf1, sem0, sem1):
        wid = lax.axis_index("s") * _NUM_SC_CORES + lax.axis_index("c")
        base = wid * b_per_w
        pltpu.sync_copy(uids_hbm.at[pl.ds(base, b_per_w)], uidx_v)
        pltpu.sync_copy(iids_hbm.at[pl.ds(base, b_per_w)], iidx_v)

        plan = [(utab_hbm, uidx_v, uout_hbm, c) for c in range(n_chunks)]
        plan += [(itab_hbm, iidx_v, iout_hbm, c) for c in range(n_chunks)]
        bufs = [buf0, buf1]
        sems = [sem0, sem1]
        copies = []
        for k, (tab, idx, _, c) in enumerate(plan):
            if k >= 2:
                copies[k - 2].wait()
                _, _, out, pc = plan[k - 2]
                pltpu.sync_copy(bufs[k % 2],
                                out.at[pl.ds(base + pc * chunk, chunk)])
            copies.append(
                pltpu.async_copy(tab.at[idx.at[pl.ds(c * chunk, chunk)]],
                                 bufs[k % 2], sems[k % 2]))
        for k in (len(plan) - 2, len(plan) - 1):
            copies[k].wait()
            _, _, out, pc = plan[k]
            pltpu.sync_copy(bufs[k % 2],
                            out.at[pl.ds(base + pc * chunk, chunk)])

    return gather_k


def _mlp_body(u_ref, i_ref, up_ref, ip_ref, w1u_ref, w1i_ref, b1_ref,
              w2_ref, b2_ref, o_ref):
    emb = w1u_ref.shape[1]
    u_sel = jnp.where(up_ref[...] == 0, u_ref[:, :emb], u_ref[:, emb:])
    i_sel = jnp.where(ip_ref[...] == 0, i_ref[:, :emb], i_ref[:, emb:])
    dn = (((1,), (1,)), ((), ()))
    h = lax.dot_general(u_sel, w1u_ref[...], dn,
                        preferred_element_type=jnp.float32)
    h = h + lax.dot_general(i_sel, w1i_ref[...], dn,
                            preferred_element_type=jnp.float32)
    h = jnp.maximum(h + b1_ref[...], 0.0)
    o = lax.dot_general(h, w2_ref[...], dn, preferred_element_type=jnp.float32)
    o_ref[...] = o + b2_ref[...]


def kernel(user_ids, item_ids, user_table, item_table, W1, b1, W2, b2):
    batch = user_ids.shape[0]
    n_rows, emb = user_table.shape
    hidden = W1.shape[0]
    width = 2 * emb

    uids = user_ids.astype(jnp.int32)
    iids = item_ids.astype(jnp.int32)

    gather_k = _make_gather(batch, width)
    u_pairs, i_pairs = gather_k(
        uids >> 1, iids >> 1,
        user_table.reshape(n_rows // 2, width),
        item_table.reshape(n_rows // 2, width))

    u_par = (uids & 1).reshape(batch, 1)
    i_par = (iids & 1).reshape(batch, 1)
    w1u = W1[:, :emb]
    w1i = W1[:, emb:]
    b1r = b1.reshape(1, hidden)
    b2r = b2.reshape(1, hidden)

    bm = 2048
    grid = (batch // bm,)
    out = pl.pallas_call(
        _mlp_body,
        grid=grid,
        in_specs=[
            pl.BlockSpec((bm, width), lambda i: (i, 0)),
            pl.BlockSpec((bm, width), lambda i: (i, 0)),
            pl.BlockSpec((bm, 1), lambda i: (i, 0)),
            pl.BlockSpec((bm, 1), lambda i: (i, 0)),
            pl.BlockSpec((hidden, emb), lambda i: (0, 0)),
            pl.BlockSpec((hidden, emb), lambda i: (0, 0)),
            pl.BlockSpec((1, hidden), lambda i: (0, 0)),
            pl.BlockSpec((hidden, hidden), lambda i: (0, 0)),
            pl.BlockSpec((1, hidden), lambda i: (0, 0)),
        ],
        out_specs=pl.BlockSpec((bm, hidden), lambda i: (i, 0)),
        out_shape=jax.ShapeDtypeStruct((batch, hidden), jnp.float32),
    )(u_pairs, i_pairs, u_par, i_par, w1u, w1i, b1r, W2, b2r)
    return out


# speedup vs baseline: 2.6165x; 1.9962x over previous
import functools

import jax
import jax.numpy as jnp
from jax import lax
from jax.experimental import pallas as pl
from jax.experimental.pallas import tpu as pltpu
from jax.experimental.pallas import tpu_sc as plsc

_NUM_SC_CORES = 2
_NUM_SC_SUBCORES = 16
_NW = _NUM_SC_CORES * _NUM_SC_SUBCORES


def _make_gather(batch: int, width: int):
    b_per_w = batch // _NW
    chunk = 128
    n_chunks = b_per_w // chunk
    mesh = plsc.VectorSubcoreMesh(
        core_axis_name="c", subcore_axis_name="s",
        num_cores=_NUM_SC_CORES, num_subcores=_NUM_SC_SUBCORES)

    @functools.partial(
        pl.kernel,
        mesh=mesh,
        out_type=[
            jax.ShapeDtypeStruct((batch, width), jnp.float32),
            jax.ShapeDtypeStruct((batch, width), jnp.float32),
        ],
        scratch_types=[
            pltpu.VMEM((b_per_w,), jnp.int32),
            pltpu.VMEM((b_per_w,), jnp.int32),
            pltpu.VMEM((chunk, width), jnp.float32),
            pltpu.VMEM((chunk, width), jnp.float32),
            pltpu.SemaphoreType.DMA,
            pltpu.SemaphoreType.DMA,
        ],
    )
    def gather_k(uids_hbm, iids_hbm, utab_hbm, itab_hbm, uout_hbm, iout_hbm,
                 uidx_v, iidx_v, buf0, buf1, sem0, sem1):
        wid = lax.axis_index("s") * _NUM_SC_CORES + lax.axis_index("c")
        base = wid * b_per_w
        pltpu.sync_copy(uids_hbm.at[pl.ds(base, b_per_w)], uidx_v)
        pltpu.sync_copy(iids_hbm.at[pl.ds(base, b_per_w)], iidx_v)

        plan = [(utab_hbm, uidx_v, uout_hbm, c) for c in range(n_chunks)]
        plan += [(itab_hbm, iidx_v, iout_hbm, c) for c in range(n_chunks)]
        bufs = [buf0, buf1]
        sems = [sem0, sem1]
        copies = []
        for k, (tab, idx, _, c) in enumerate(plan):
            if k >= 2:
                copies[k - 2].wait()
                _, _, out, pc = plan[k - 2]
                pltpu.sync_copy(bufs[k % 2],
                                out.at[pl.ds(base + pc * chunk, chunk)])
            copies.append(
                pltpu.async_copy(tab.at[idx.at[pl.ds(c * chunk, chunk)]],
                                 bufs[k % 2], sems[k % 2]))
        for k in (len(plan) - 2, len(plan) - 1):
            copies[k].wait()
            _, _, out, pc = plan[k]
            pltpu.sync_copy(bufs[k % 2],
                            out.at[pl.ds(base + pc * chunk, chunk)])

    return gather_k


_BN = 8192


def _repack_body(t_ref, o_ref):
    x = t_ref[...]
    half = _BN // 2
    o_ref[:, : x.shape[0]] = x[:, :half].T
    o_ref[:, x.shape[0]:] = x[:, half:].T


def _make_repack(n_rows: int, emb: int):
    width = 2 * emb
    n_blocks = (n_rows + _BN - 1) // _BN

    def repack(table_t):
        return pl.pallas_call(
            _repack_body,
            grid=(n_blocks,),
            in_specs=[pl.BlockSpec((emb, _BN), lambda i: (0, i))],
            out_specs=pl.BlockSpec((_BN // 2, width), lambda i: (i, 0)),
            out_shape=jax.ShapeDtypeStruct((n_blocks * (_BN // 2), width),
                                           jnp.float32),
        )(table_t)

    return repack


def _mlp_body(u_ref, i_ref, up_ref, ip_ref, w1u_ref, w1i_ref, b1_ref,
              w2_ref, b2_ref, o_ref):
    emb = w1u_ref.shape[1]
    u_sel = jnp.where(up_ref[...] == 0, u_ref[:, :emb], u_ref[:, emb:])
    i_sel = jnp.where(ip_ref[...] == 0, i_ref[:, :emb], i_ref[:, emb:])
    dn = (((1,), (1,)), ((), ()))
    h = lax.dot_general(u_sel, w1u_ref[...], dn,
                        preferred_element_type=jnp.float32)
    h = h + lax.dot_general(i_sel, w1i_ref[...], dn,
                            preferred_element_type=jnp.float32)
    h = jnp.maximum(h + b1_ref[...], 0.0)
    o = lax.dot_general(h, w2_ref[...], dn, preferred_element_type=jnp.float32)
    o_ref[...] = o + b2_ref[...]


def kernel(user_ids, item_ids, user_table, item_table, W1, b1, W2, b2):
    batch = user_ids.shape[0]
    n_rows, emb = user_table.shape
    hidden = W1.shape[0]
    width = 2 * emb

    uids = user_ids.astype(jnp.int32)
    iids = item_ids.astype(jnp.int32)

    repack = _make_repack(n_rows, emb)
    u_packed = repack(user_table.T)
    i_packed = repack(item_table.T)

    half = _BN // 2
    def packed_idx(ids):
        return (ids // _BN) * half + (ids % _BN) % half

    gather_k = _make_gather(batch, width)
    u_pairs, i_pairs = gather_k(packed_idx(uids), packed_idx(iids),
                                u_packed, i_packed)

    u_par = ((uids % _BN) >= half).astype(jnp.int32).reshape(batch, 1)
    i_par = ((iids % _BN) >= half).astype(jnp.int32).reshape(batch, 1)
    w1u = W1[:, :emb]
    w1i = W1[:, emb:]
    b1r = b1.reshape(1, hidden)
    b2r = b2.reshape(1, hidden)

    bm = 2048
    grid = (batch // bm,)
    out = pl.pallas_call(
        _mlp_body,
        grid=grid,
        in_specs=[
            pl.BlockSpec((bm, width), lambda i: (i, 0)),
            pl.BlockSpec((bm, width), lambda i: (i, 0)),
            pl.BlockSpec((bm, 1), lambda i: (i, 0)),
            pl.BlockSpec((bm, 1), lambda i: (i, 0)),
            pl.BlockSpec((hidden, emb), lambda i: (0, 0)),
            pl.BlockSpec((hidden, emb), lambda i: (0, 0)),
            pl.BlockSpec((1, hidden), lambda i: (0, 0)),
            pl.BlockSpec((hidden, hidden), lambda i: (0, 0)),
            pl.BlockSpec((1, hidden), lambda i: (0, 0)),
        ],
        out_specs=pl.BlockSpec((bm, hidden), lambda i: (i, 0)),
        out_shape=jax.ShapeDtypeStruct((batch, hidden), jnp.float32),
    )(u_pairs, i_pairs, u_par, i_par, w1u, w1i, b1r, W2, b2r)
    return out
